# baseline (speedup 1.0000x reference)
import functools

import jax
import jax.numpy as jnp
from jax import lax
from jax.experimental import pallas as pl
from jax.experimental.pallas import tpu as pltpu
from jax.experimental.pallas import tpu_sc as plsc

NC, NS = 2, 16
NW = NC * NS
ROW = 128
K = 8
HID = 64


@functools.lru_cache(maxsize=None)
def _make_kernel(n_rows: int):
    rows_per_w = n_rows // NW
    groups = rows_per_w // K
    mesh = plsc.VectorSubcoreMesh(core_axis_name="c", subcore_axis_name="s")

    @functools.partial(
        pl.kernel,
        out_type=jax.ShapeDtypeStruct((n_rows, ROW, HID), jnp.float32),
        mesh=mesh,
        scratch_types=[
            pltpu.VMEM((K, ROW), jnp.int32),
            pltpu.VMEM((K, ROW, HID), jnp.float32),
            pltpu.SemaphoreType.DMA,
        ],
        compiler_params=pltpu.CompilerParams(use_tc_tiling_on_sc=False),
    )
    def k(ids_hbm, table_hbm, out_hbm, idx_v, rows_v, sem):
        wid = lax.axis_index("s") * NC + lax.axis_index("c")
        row_base = wid * rows_per_w

        @pl.loop(0, groups)
        def _group(g):
            r0 = row_base + g * K
            pltpu.sync_copy(ids_hbm.at[pl.ds(r0, K)], idx_v)
            cps = [
                pltpu.async_copy(table_hbm.at[idx_v.at[j]], rows_v.at[j], sem)
                for j in range(K)
            ]
            for cp in cps:
                cp.wait()
            pltpu.sync_copy(rows_v, out_hbm.at[pl.ds(r0, K)])

    return k


def kernel(input_ids, wte):
    B, L = input_ids.shape
    n = B * L
    n_rows = n // ROW
    ids = input_ids.reshape(n_rows, ROW).astype(jnp.int32)
    out = _make_kernel(n_rows)(ids, wte)
    return out.reshape(B, L, HID)
